# baseline (speedup 1.0000x reference)
import functools

import jax
import jax.numpy as jnp
from jax import lax
from jax.experimental import pallas as pl
from jax.experimental.pallas import tpu as pltpu
from jax.experimental.pallas import tpu_sc as plsc

VOCAB_SIZE = 1000000
EMB = 64
BATCH = 16384
C = 5

NUM_CORES = 2
NUM_SUBCORES = 16
NW = NUM_CORES * NUM_SUBCORES
B_PER_W = BATCH // NW
CB = 128
N_CHUNKS = B_PER_W // CB
NG = CB // 16


def _body(tgt_hbm, ctx_hbm, ttab_hbm, ctab_hbm, out_hbm,
          idxw_v, idxc_v, hiw_v, hic_v, low_v, loc_v,
          w_rows, c_rows, out_v, sem):
    wid = lax.axis_index("s") * NUM_CORES + lax.axis_index("c")
    base = wid * B_PER_W
    iota = lax.iota(jnp.int32, 16)

    for k in range(N_CHUNKS):
        start = base + k * CB
        pltpu.sync_copy(tgt_hbm.at[pl.ds(start, CB)], idxw_v)
        pltpu.sync_copy(ctx_hbm.at[pl.ds(start * C, CB * C)], idxc_v)

        def split_w(i, carry):
            x = idxw_v[pl.ds(i * 16, 16)]
            hiw_v[pl.ds(i * 16, 16)] = x >> 1
            low_v[pl.ds(i * 16, 16)] = (x & 1) * EMB
            return carry

        lax.fori_loop(0, CB // 16, split_w, 0)

        def split_c(i, carry):
            x = idxc_v[pl.ds(i * 16, 16)]
            hic_v[pl.ds(i * 16, 16)] = x >> 1
            loc_v[pl.ds(i * 16, 16)] = (x & 1) * EMB
            return carry

        lax.fori_loop(0, CB * C // 16, split_c, 0)

        g1 = pltpu.async_copy(ttab_hbm.at[hiw_v], w_rows, sem)
        g2 = pltpu.async_copy(ctab_hbm.at[hic_v], c_rows, sem)
        g1.wait()
        g2.wait()

        def gbody(g, carry):
            rw = g * 16 + iota
            colw0 = plsc.load_gather(low_v, [rw])
            rc = [rw * C + c for c in range(C)]
            colc0 = [plsc.load_gather(loc_v, [rc[c]]) for c in range(C)]

            def ebody(e, accs):
                wt = plsc.load_gather(w_rows, [rw, colw0 + e])
                return tuple(
                    accs[c] + wt * plsc.load_gather(c_rows, [rc[c], colc0[c] + e])
                    for c in range(C)
                )

            zeros = jnp.zeros((16,), jnp.float32)
            accs = lax.fori_loop(0, EMB, ebody, (zeros,) * C, unroll=4)
            for c in range(C):
                plsc.store_scatter(out_v, [rc[c]], accs[c])
            return carry

        lax.fori_loop(0, NG, gbody, 0)
        pltpu.sync_copy(out_v, out_hbm.at[pl.ds(start * C, CB * C)])


def kernel(target, context, target_table, context_table):
    tgt = target.reshape(BATCH).astype(jnp.int32)
    ctx = context.reshape(BATCH * C).astype(jnp.int32)
    ttab = target_table.reshape(VOCAB_SIZE // 2, 2 * EMB)
    ctab = context_table.reshape(VOCAB_SIZE // 2, 2 * EMB)

    mesh = plsc.VectorSubcoreMesh(core_axis_name="c", subcore_axis_name="s")
    run = functools.partial(
        pl.kernel,
        mesh=mesh,
        compiler_params=pltpu.CompilerParams(needs_layout_passes=False),
        out_type=jax.ShapeDtypeStruct((BATCH * C,), jnp.float32),
        scratch_types=[
            pltpu.VMEM((CB,), jnp.int32),
            pltpu.VMEM((CB * C,), jnp.int32),
            pltpu.VMEM((CB,), jnp.int32),
            pltpu.VMEM((CB * C,), jnp.int32),
            pltpu.VMEM((CB,), jnp.int32),
            pltpu.VMEM((CB * C,), jnp.int32),
            pltpu.VMEM((CB, 2 * EMB), jnp.float32),
            pltpu.VMEM((CB * C, 2 * EMB), jnp.float32),
            pltpu.VMEM((CB * C,), jnp.float32),
            pltpu.SemaphoreType.DMA,
        ],
    )(_body)
    out = run(tgt, ctx, ttab, ctab)
    return out.reshape(BATCH, C)


# speedup vs baseline: 1.2876x; 1.2876x over previous
import functools

import jax
import jax.numpy as jnp
from jax import lax
from jax.experimental import pallas as pl
from jax.experimental.pallas import tpu as pltpu
from jax.experimental.pallas import tpu_sc as plsc

VOCAB_SIZE = 1000000
EMB = 64
BATCH = 16384
C = 5

NUM_CORES = 2
NUM_SUBCORES = 16
NW = NUM_CORES * NUM_SUBCORES
B_PER_W = BATCH // NW
CB = 128
N_CHUNKS = B_PER_W // CB

SB_STRIDE = 17
N_OUT_GROUPS = CB * C // 16


def _body(tgt_hbm, ctx_hbm, tab_hbm, out_hbm,
          idx_v, cidx_v, w_rows, c_rows, sbuf, out_v, sem):
    wid = lax.axis_index("s") * NUM_CORES + lax.axis_index("c")
    base = wid * B_PER_W
    iota = lax.iota(jnp.int32, 16)
    iota_sb = iota * SB_STRIDE

    for k in range(N_CHUNKS):
        start = base + k * CB
        pltpu.sync_copy(tgt_hbm.at[pl.ds(start, CB)], idx_v)
        pltpu.sync_copy(ctx_hbm.at[pl.ds(start * C, CB * C)], cidx_v)
        g1 = pltpu.async_copy(tab_hbm.at[idx_v], w_rows, sem)
        g2 = pltpu.async_copy(tab_hbm.at[cidx_v], c_rows, sem)
        g1.wait()
        g2.wait()

        def bbody(b, carry):
            w = [w_rows[b, pl.ds(16 * q, 16)] for q in range(4)]
            for c in range(C):
                cc = [c_rows[b * C + c, pl.ds(64 + 16 * q, 16)] for q in range(4)]
                s = (w[0] * cc[0] + w[1] * cc[1]) + (w[2] * cc[2] + w[3] * cc[3])
                sbuf[pl.ds((b * C + c) * SB_STRIDE, 16)] = s
            return carry

        lax.fori_loop(0, CB, bbody, 0)

        def gbody(og, carry):
            gb = og * (16 * SB_STRIDE)
            acc = plsc.load_gather(sbuf, [iota_sb + gb])
            for j in range(1, 16):
                acc = acc + plsc.load_gather(sbuf, [iota_sb + (gb + j)])
            out_v[pl.ds(og * 16, 16)] = acc
            return carry

        lax.fori_loop(0, N_OUT_GROUPS, gbody, 0)
        pltpu.sync_copy(out_v, out_hbm.at[pl.ds(start * C, CB * C)])


def kernel(target, context, target_table, context_table):
    tgt = target.reshape(BATCH).astype(jnp.int32)
    ctx = context.reshape(BATCH * C).astype(jnp.int32)
    tab = jnp.concatenate([target_table, context_table], axis=1)

    mesh = plsc.VectorSubcoreMesh(core_axis_name="c", subcore_axis_name="s")
    run = functools.partial(
        pl.kernel,
        mesh=mesh,
        compiler_params=pltpu.CompilerParams(needs_layout_passes=False),
        out_type=jax.ShapeDtypeStruct((BATCH * C,), jnp.float32),
        scratch_types=[
            pltpu.VMEM((CB,), jnp.int32),
            pltpu.VMEM((CB * C,), jnp.int32),
            pltpu.VMEM((CB, 128), jnp.float32),
            pltpu.VMEM((CB * C, 128), jnp.float32),
            pltpu.VMEM((CB * C * SB_STRIDE,), jnp.float32),
            pltpu.VMEM((CB * C,), jnp.float32),
            pltpu.SemaphoreType.DMA,
        ],
    )(_body)
    out = run(tgt, ctx, tab)
    return out.reshape(BATCH, C)
